# baseline (speedup 1.0000x reference)
import functools

import jax
import jax.numpy as jnp
from jax import lax
from jax.experimental import pallas as pl
from jax.experimental.pallas import tpu as pltpu
from jax.experimental.pallas import tpu_sc as plsc


def _sc_gather(table, idx_flat, n_total, d):
    info = plsc.get_sparse_core_info()
    nc, ns = info.num_cores, info.num_subcores
    nw = nc * ns
    n_per_w = n_total // nw
    n_chunks = 4
    ch = n_per_w // n_chunks
    assert ch * n_chunks == n_per_w and ch % 8 == 0

    mesh = plsc.VectorSubcoreMesh(core_axis_name="c", subcore_axis_name="s")

    @functools.partial(
        pl.kernel,
        mesh=mesh,
        out_type=jax.ShapeDtypeStruct((n_total, d), jnp.float32),
        scratch_types=[
            pltpu.VMEM((ch,), jnp.int32),
            pltpu.VMEM((ch, d), jnp.float32),
            pltpu.SemaphoreType.DMA,
        ],
        compiler_params=pltpu.CompilerParams(use_tc_tiling_on_sc=False),
    )
    def k(table_hbm, idx_hbm, out_hbm, idx_v, rows_v, sem):
        wid = lax.axis_index("s") * nc + lax.axis_index("c")
        base = wid * n_per_w
        for c in range(n_chunks):
            b0 = base + c * ch
            pltpu.sync_copy(idx_hbm.at[pl.ds(b0, ch)], idx_v)
            pltpu.async_copy(table_hbm.at[idx_v], rows_v, sem).wait()
            pltpu.sync_copy(rows_v, out_hbm.at[pl.ds(b0, ch)])

    return k(table, idx_flat)


def _tc_num(x_num, w, b, block_b=2048):
    bsz, f = x_num.shape
    d = w.shape[1]

    def body(x_ref, w_ref, b_ref, o_ref):
        o_ref[...] = x_ref[...][:, :, None] * w_ref[...][None] + b_ref[...][None]

    return pl.pallas_call(
        body,
        grid=(bsz // block_b,),
        in_specs=[
            pl.BlockSpec((block_b, f), lambda i: (i, 0)),
            pl.BlockSpec((f, d), lambda i: (0, 0)),
            pl.BlockSpec((f, d), lambda i: (0, 0)),
        ],
        out_specs=pl.BlockSpec((block_b, f, d), lambda i: (i, 0, 0)),
        out_shape=jax.ShapeDtypeStruct((bsz, f, d), jnp.float32),
    )(x_num, w, b)


def kernel(x_num, x_cat, num_weight, num_bias, cat_table, category_offsets):
    bsz, fc = x_cat.shape
    d = cat_table.shape[1]
    idx_flat = (
        x_cat.astype(jnp.int32) + category_offsets.astype(jnp.int32)[None]
    ).reshape(-1)
    cat_tok = _sc_gather(cat_table, idx_flat, bsz * fc, d).reshape(bsz, fc, d)
    num_tok = _tc_num(x_num, num_weight, num_bias)
    return jnp.concatenate([num_tok, cat_tok], axis=1)


# speedup vs baseline: 1.1402x; 1.1402x over previous
import functools

import jax
import jax.numpy as jnp
from jax import lax
from jax.experimental import pallas as pl
from jax.experimental.pallas import tpu as pltpu
from jax.experimental.pallas import tpu_sc as plsc

_LANES = 16


def _sc_gather(table, idx_raw, off_pattern, n_total, d, ch):
    info = plsc.get_sparse_core_info()
    nc, ns = info.num_cores, info.num_subcores
    nw = nc * ns
    n_per_w = n_total // nw
    n_chunks = n_per_w // ch
    assert ch * n_chunks == n_per_w and ch % 8 == 0

    mesh = plsc.VectorSubcoreMesh(core_axis_name="c", subcore_axis_name="s")

    @functools.partial(
        pl.kernel,
        mesh=mesh,
        out_type=jax.ShapeDtypeStruct((n_total, 128), jnp.float32),
        scratch_types=[
            pltpu.VMEM((ch,), jnp.int32),
            pltpu.VMEM((ch, d), jnp.float32),
            pltpu.VMEM((ch,), jnp.int32),
            pltpu.SemaphoreType.DMA,
        ],
        compiler_params=pltpu.CompilerParams(use_tc_tiling_on_sc=False),
    )
    def k(table_hbm, idx_hbm, offp_hbm, out_hbm, idx_v, rows_v, off_v, sem):
        wid = lax.axis_index("s") * nc + lax.axis_index("c")
        base = wid * n_per_w
        pltpu.sync_copy(offp_hbm, off_v)
        for c in range(n_chunks):
            b0 = base + c * ch
            pltpu.sync_copy(idx_hbm.at[pl.ds(b0, ch)], idx_v)

            def add_off(i, carry):
                s = pl.ds(i * _LANES, _LANES)
                idx_v[s] = idx_v[s] + off_v[s]
                return carry

            lax.fori_loop(0, ch // _LANES, add_off, 0)
            pltpu.async_copy(table_hbm.at[idx_v], rows_v, sem).wait()
            pltpu.sync_copy(rows_v, out_hbm.at[pl.ds(b0, ch), pl.ds(0, d)])

    return k(table, idx_raw, off_pattern)


def _tc_assemble(x_num, w, b, cat_stage, fc, block_b=256):
    bsz, f = x_num.shape
    d = w.shape[1]

    def body(x_ref, w_ref, b_ref, cat_ref, o_ref):
        o_ref[:, :f, :] = x_ref[...][:, :, None] * w_ref[...][None] + b_ref[...][None]
        o_ref[:, f:, :] = cat_ref[:, :d].reshape(block_b, fc, d)

    return pl.pallas_call(
        body,
        grid=(bsz // block_b,),
        in_specs=[
            pl.BlockSpec((block_b, f), lambda i: (i, 0)),
            pl.BlockSpec((f, d), lambda i: (0, 0)),
            pl.BlockSpec((f, d), lambda i: (0, 0)),
            pl.BlockSpec((block_b * fc, 128), lambda i: (i, 0)),
        ],
        out_specs=pl.BlockSpec((block_b, f + fc, d), lambda i: (i, 0, 0)),
        out_shape=jax.ShapeDtypeStruct((bsz, f + fc, d), jnp.float32),
    )(x_num, w, b, cat_stage)


def kernel(x_num, x_cat, num_weight, num_bias, cat_table, category_offsets):
    bsz, fc = x_cat.shape
    d = cat_table.shape[1]
    idx_raw = x_cat.astype(jnp.int32).reshape(-1)
    ch = 128 * fc
    off_pattern = jnp.tile(category_offsets.astype(jnp.int32), ch // fc)
    cat_stage = _sc_gather(cat_table, idx_raw, off_pattern, bsz * fc, d, ch)
    return _tc_assemble(x_num, num_weight, num_bias, cat_stage, fc)
